# Initial kernel scaffold; baseline (speedup 1.0000x reference)
import jax
import jax.numpy as jnp
from jax.experimental import pallas as pl


def kernel(words, W_w, W_c):
    raise NotImplementedError("write your pallas kernel here")


# speedup vs baseline: 1.6977x; 1.6977x over previous
import jax
import jax.numpy as jnp
from jax import lax
from jax.experimental import pallas as pl
from jax.experimental.pallas import tpu as pltpu
from jax.experimental.pallas import tpu_sc as plsc

DIM = 64
NC, NS = 2, 16
NW = NC * NS
CH = 128


def _gather_body(idx_hbm, table_hbm, out_hbm, idx_v, rows_v, gsem):
    wid = lax.axis_index("s") * NC + lax.axis_index("c")
    n_ch = idx_hbm.shape[1]
    base = wid * (n_ch * CH)
    pltpu.sync_copy(idx_hbm.at[wid], idx_v)

    def step(j, carry):
        pltpu.async_copy(table_hbm.at[idx_v.at[j]], rows_v, gsem).wait()
        pltpu.sync_copy(rows_v, out_hbm.at[pl.ds(base + j * CH, CH)])
        return carry

    lax.fori_loop(0, n_ch, step, 0)


def kernel(words, W_w, W_c):
    B = words.shape[0] * words.shape[1]
    n_ch = B // (NW * CH)
    idx = words.reshape(NW, n_ch, CH)
    out = pl.kernel(
        _gather_body,
        out_type=jax.ShapeDtypeStruct((B, DIM), jnp.float32),
        mesh=plsc.VectorSubcoreMesh(core_axis_name="c", subcore_axis_name="s"),
        scratch_types=[
            pltpu.VMEM((n_ch, CH), jnp.int32),
            pltpu.VMEM((CH, DIM), jnp.float32),
            pltpu.SemaphoreType.DMA,
        ],
        compiler_params=pltpu.CompilerParams(use_tc_tiling_on_sc=False),
    )(idx, W_w)
    return out.reshape(words.shape[0], words.shape[1], DIM)


# speedup vs baseline: 1.8750x; 1.1045x over previous
import jax
import jax.numpy as jnp
from jax import lax
from jax.experimental import pallas as pl
from jax.experimental.pallas import tpu as pltpu
from jax.experimental.pallas import tpu_sc as plsc

DIM = 64
NC, NS = 2, 16
NW = NC * NS
CH = 128
K = 5


def _gather_body(idx_hbm, table_hbm, out_hbm, idx_v, rows_v, gs, os0, os1):
    wid = lax.axis_index("s") * NC + lax.axis_index("c")
    n_ch = idx_hbm.shape[1]
    base = wid * (n_ch * CH)
    oss = (os0, os1)
    pltpu.sync_copy(idx_hbm.at[wid], idx_v)

    def do_block(o, d):
        descs = [
            pltpu.async_copy(
                table_hbm.at[idx_v.at[o * K + j]],
                rows_v.at[d, pl.ds(j * CH, CH)],
                gs,
            )
            for j in range(K)
        ]
        for desc in descs:
            desc.wait()
        pltpu.async_copy(
            rows_v.at[d],
            out_hbm.at[pl.ds(base + o * K * CH, K * CH)],
            oss[d],
        )

    def outer(p, carry):
        for dd in (0, 1):
            @pl.when(p >= 1)
            def _drain():
                pltpu.make_async_copy(
                    out_hbm.at[pl.ds(0, K * CH)], rows_v.at[dd], oss[dd]
                ).wait()
            do_block(2 * p + dd, dd)
        return carry

    n_blk = n_ch // K
    lax.fori_loop(0, n_blk // 2, outer, 0)
    for dd in (0, 1):
        pltpu.make_async_copy(
            out_hbm.at[pl.ds(0, K * CH)], rows_v.at[dd], oss[dd]
        ).wait()


def kernel(words, W_w, W_c):
    B = words.shape[0] * words.shape[1]
    n_ch = B // (NW * CH)
    idx = words.reshape(NW, n_ch, CH)
    out = pl.kernel(
        _gather_body,
        out_type=jax.ShapeDtypeStruct((B, DIM), jnp.float32),
        mesh=plsc.VectorSubcoreMesh(core_axis_name="c", subcore_axis_name="s"),
        scratch_types=[
            pltpu.VMEM((n_ch, CH), jnp.int32),
            pltpu.VMEM((2, K * CH, DIM), jnp.float32),
            pltpu.SemaphoreType.DMA,
            pltpu.SemaphoreType.DMA,
            pltpu.SemaphoreType.DMA,
        ],
        compiler_params=pltpu.CompilerParams(use_tc_tiling_on_sc=False),
    )(idx, W_w)
    return out.reshape(words.shape[0], words.shape[1], DIM)
